# baseline (speedup 1.0000x reference)
import functools

import jax
import jax.numpy as jnp
from jax import lax
from jax.experimental import pallas as pl

SEQ = 2048
DIM = 2048
BATCH = 4
ROWS_PER_TILE = 512


def _logits_body(x_ref, w_ref, out_ref):
    xt = x_ref[...].astype(jnp.bfloat16).astype(jnp.float32)
    w = w_ref[...].astype(jnp.bfloat16).astype(jnp.float32)
    out_ref[...] = jnp.sum(xt * w, axis=1, keepdims=True)


def _block_body(x_ref, w_ref, lcol_ref, lrow_ref, out_ref, *, k, rows, seq):
    s = pl.program_id(1)
    xt = x_ref[0]
    acc = lax.dot_general(xt.astype(jnp.bfloat16), w_ref[...],
                          (((1,), (1,)), ((), ())),
                          preferred_element_type=jnp.float32)
    lc = lcol_ref[0]
    lr = lrow_ref[0]
    i_idx = lax.broadcasted_iota(jnp.int32, (rows, seq), 0) + s * rows
    j_idx = lax.broadcasted_iota(jnp.int32, (rows, seq), 1)
    beats = (lr > lc) | ((lr == lc) & (j_idx < i_idx))
    cnt = jnp.sum(beats.astype(jnp.float32), axis=1, keepdims=True)
    out_ref[0] = jnp.where(cnt < k, acc, xt)


def kernel(x, W_block, W_router):
    B, S, D = x.shape
    k = int(S * 0.5)
    xf = x.reshape(B * S, D)
    rows = ROWS_PER_TILE
    n_tiles = (B * S) // rows

    logits = pl.pallas_call(
        _logits_body,
        grid=(n_tiles,),
        in_specs=[
            pl.BlockSpec((rows, D), lambda i: (i, 0)),
            pl.BlockSpec((1, D), lambda i: (0, 0)),
        ],
        out_specs=pl.BlockSpec((rows, 1), lambda i: (i, 0)),
        out_shape=jax.ShapeDtypeStruct((B * S, 1), jnp.float32),
    )(xf, W_router)

    wb = W_block.astype(jnp.bfloat16)
    out = pl.pallas_call(
        functools.partial(_block_body, k=k, rows=rows, seq=S),
        grid=(B, S // rows),
        in_specs=[
            pl.BlockSpec((1, rows, D), lambda b, s: (b, s, 0)),
            pl.BlockSpec((D, D), lambda b, s: (0, 0)),
            pl.BlockSpec((1, rows, 1), lambda b, s: (b, s, 0)),
            pl.BlockSpec((1, 1, S), lambda b, s: (b, 0, 0)),
        ],
        out_specs=pl.BlockSpec((1, rows, D), lambda b, s: (b, s, 0)),
        out_shape=jax.ShapeDtypeStruct((B, S, D), jnp.float32),
    )(x, wb, logits.reshape(B, S, 1), logits.reshape(B, 1, S))

    return out


# speedup vs baseline: 1.0011x; 1.0011x over previous
import functools

import jax
import jax.numpy as jnp
from jax import lax
from jax.experimental import pallas as pl

SEQ = 2048
DIM = 2048
BATCH = 4
ROWS_PER_TILE = 1024


def _logits_body(x_ref, w_ref, out_ref):
    xt = x_ref[...].astype(jnp.bfloat16).astype(jnp.float32)
    w = w_ref[...].astype(jnp.bfloat16).astype(jnp.float32)
    out_ref[...] = jnp.sum(xt * w, axis=1, keepdims=True)


def _block_body(x_ref, w_ref, lcol_ref, lrow_ref, out_ref, *, k, rows, seq):
    s = pl.program_id(1)
    xt = x_ref[0]
    acc = lax.dot_general(xt.astype(jnp.bfloat16), w_ref[...],
                          (((1,), (1,)), ((), ())),
                          preferred_element_type=jnp.float32)
    lc = lcol_ref[0]
    lr = lrow_ref[0]
    i_idx = lax.broadcasted_iota(jnp.int32, (rows, seq), 0) + s * rows
    j_idx = lax.broadcasted_iota(jnp.int32, (rows, seq), 1)
    beats = (lr > lc) | ((lr == lc) & (j_idx < i_idx))
    ones = jnp.ones((seq, 1), jnp.bfloat16)
    cnt = jnp.dot(beats.astype(jnp.bfloat16), ones,
                  preferred_element_type=jnp.float32)
    out_ref[0] = jnp.where(cnt < k, acc, xt)


def kernel(x, W_block, W_router):
    B, S, D = x.shape
    k = int(S * 0.5)
    xf = x.reshape(B * S, D)
    rows = ROWS_PER_TILE
    n_tiles = (B * S) // rows

    logits = pl.pallas_call(
        _logits_body,
        grid=(n_tiles,),
        in_specs=[
            pl.BlockSpec((rows, D), lambda i: (i, 0)),
            pl.BlockSpec((1, D), lambda i: (0, 0)),
        ],
        out_specs=pl.BlockSpec((rows, 1), lambda i: (i, 0)),
        out_shape=jax.ShapeDtypeStruct((B * S, 1), jnp.float32),
    )(xf, W_router)

    wb = W_block.astype(jnp.bfloat16)
    out = pl.pallas_call(
        functools.partial(_block_body, k=k, rows=rows, seq=S),
        grid=(B, S // rows),
        in_specs=[
            pl.BlockSpec((1, rows, D), lambda b, s: (b, s, 0)),
            pl.BlockSpec((D, D), lambda b, s: (0, 0)),
            pl.BlockSpec((1, rows, 1), lambda b, s: (b, s, 0)),
            pl.BlockSpec((1, 1, S), lambda b, s: (b, 0, 0)),
        ],
        out_specs=pl.BlockSpec((1, rows, D), lambda b, s: (b, s, 0)),
        out_shape=jax.ShapeDtypeStruct((B, S, D), jnp.float32),
    )(x, wb, logits.reshape(B, S, 1), logits.reshape(B, 1, S))

    return out
